# baseline (speedup 1.0000x reference)
import jax
import jax.numpy as jnp
from jax import lax
from jax.experimental import pallas as pl
from jax.experimental.pallas import tpu as pltpu
from jax.experimental.pallas import tpu_sc as plsc

_N_FACTORS = 64
_BATCH = 16384
_NC, _NS, _L = 2, 16, 16
_NW = _NC * _NS
_BPW = _BATCH // _NW
_WAVE = 256
_FEAT_BLK = 2048


def _feat_body(gb_ref, uf_ref, if_ref, wu_ref, wi_ref, out_ref):
    acc = jnp.sum(uf_ref[...] * wu_ref[...], axis=1)
    acc = acc + jnp.sum(if_ref[...] * wi_ref[...], axis=1)
    out_ref[...] = acc + gb_ref[0]


def _feat_call(global_bias, user_features, item_features, w_u, w_i):
    batch, fdim = user_features.shape
    grid = batch // _FEAT_BLK
    return pl.pallas_call(
        _feat_body,
        grid=(grid,),
        in_specs=[
            pl.BlockSpec(memory_space=pltpu.SMEM),
            pl.BlockSpec((_FEAT_BLK, fdim), lambda i: (i, 0)),
            pl.BlockSpec((_FEAT_BLK, fdim), lambda i: (i, 0)),
            pl.BlockSpec((1, fdim), lambda i: (0, 0)),
            pl.BlockSpec((1, fdim), lambda i: (0, 0)),
        ],
        out_specs=pl.BlockSpec((_FEAT_BLK,), lambda i: (i,)),
        out_shape=jax.ShapeDtypeStruct((batch,), jnp.float32),
    )(global_bias, user_features, item_features, w_u, w_i)


def _sc_body(uid_hbm, iid_hbm, uemb_hbm, iemb_hbm, ub_hbm, ib_hbm, feat_hbm,
             out_hbm, uidx_v, iidx_v, ucols_v, icols_v,
             ub_v, ib_v, feat_v, out_v, sem, bsem):
    wid = lax.axis_index("s") * _NC + lax.axis_index("c")
    base = wid * _BPW
    pltpu.sync_copy(uid_hbm.at[pl.ds(base, _BPW)], uidx_v)
    pltpu.sync_copy(iid_hbm.at[pl.ds(base, _BPW)], iidx_v)
    pltpu.sync_copy(feat_hbm.at[pl.ds(base, _BPW)], feat_v)
    cb1 = pltpu.async_copy(ub_hbm.at[uidx_v], ub_v, bsem)
    cb2 = pltpu.async_copy(ib_hbm.at[iidx_v], ib_v, bsem)

    cb1.wait()
    cb2.wait()

    def wave(w, carry):
        wrow = w * _WAVE

        def fetch(q, carry2):
            uvec = uidx_v[pl.ds(wrow + q * _L, _L)]
            ivec = iidx_v[pl.ds(wrow + q * _L, _L)]
            for r in range(_L):
                c = q * _L + r
                pltpu.async_copy(uemb_hbm.at[uvec[r]],
                                 ucols_v.at[c, pl.ds(0, _N_FACTORS)], sem)
                pltpu.async_copy(iemb_hbm.at[ivec[r]],
                                 icols_v.at[c, pl.ds(0, _N_FACTORS)], sem)
            return carry2

        lax.fori_loop(0, _WAVE // _L, fetch, 0)

        def drain(g, carry2):
            pltpu.make_async_copy(
                uemb_hbm.at[0],
                ucols_v.at[0, pl.ds(0, _N_FACTORS)], sem).wait()
            pltpu.make_async_copy(
                iemb_hbm.at[0],
                icols_v.at[0, pl.ds(0, _N_FACTORS)], sem).wait()
            return carry2

        lax.fori_loop(0, _WAVE, drain, 0)

        def group(g, carry3):
            boff = g * _L
            rows = boff + lax.iota(jnp.int32, _L)
            acc = feat_v[pl.ds(wrow + boff, _L)]
            acc = acc + ub_v[pl.ds(wrow + boff, _L)]
            acc = acc + ib_v[pl.ds(wrow + boff, _L)]
            for k in range(_N_FACTORS):
                cols = jnp.full((_L,), k, jnp.int32)
                u = plsc.load_gather(ucols_v, [rows, cols])
                i = plsc.load_gather(icols_v, [rows, cols])
                acc = acc + u * i
            out_v[pl.ds(wrow + boff, _L)] = acc
            return carry3

        lax.fori_loop(0, _WAVE // _L, group, 0)
        return carry

    lax.fori_loop(0, _BPW // _WAVE, wave, 0)
    pltpu.sync_copy(out_v, out_hbm.at[pl.ds(base, _BPW)])


def _sc_call(user_ids, item_ids, user_embed, item_embed, user_bias,
             item_bias, feat):
    mesh = plsc.VectorSubcoreMesh(
        core_axis_name="c", subcore_axis_name="s",
        num_cores=_NC, num_subcores=_NS)
    run = pl.kernel(
        _sc_body,
        out_type=jax.ShapeDtypeStruct((_BATCH,), jnp.float32),
        mesh=mesh,
        compiler_params=pltpu.CompilerParams(needs_layout_passes=False),
        scratch_types=[
            pltpu.VMEM((_BPW,), jnp.int32),
            pltpu.VMEM((_BPW,), jnp.int32),
            pltpu.VMEM((_WAVE, _N_FACTORS), jnp.float32),
            pltpu.VMEM((_WAVE, _N_FACTORS), jnp.float32),
            pltpu.VMEM((_BPW,), jnp.float32),
            pltpu.VMEM((_BPW,), jnp.float32),
            pltpu.VMEM((_BPW,), jnp.float32),
            pltpu.VMEM((_BPW,), jnp.float32),
            pltpu.SemaphoreType.DMA,
            pltpu.SemaphoreType.DMA,
        ],
    )
    return run(user_ids, item_ids, user_embed, item_embed, user_bias,
               item_bias, feat)


def kernel(user_ids, item_ids, user_features, item_features, user_embed,
           item_embed, user_bias, item_bias, global_bias, w_u, w_i):
    feat = _feat_call(global_bias, user_features, item_features, w_u, w_i)
    return _sc_call(user_ids, item_ids, user_embed, item_embed,
                    user_bias.T.reshape(-1), item_bias.T.reshape(-1), feat)
